# baseline (speedup 1.0000x reference)
import functools

import jax
import jax.numpy as jnp
from jax import lax
from jax.experimental import pallas as pl
from jax.experimental.pallas import tpu as pltpu
from jax.experimental.pallas import tpu_sc as plsc

HID = 64
LPAD = 56
NBUF = 8


def _gather_grid(b: int, l: int, hid: int):
    info = plsc.get_sparse_core_info()
    num_workers = info.num_cores * info.num_subcores
    bp_w = b // num_workers
    n_outer = bp_w // NBUF
    mesh = plsc.VectorSubcoreMesh(core_axis_name="c", subcore_axis_name="s")

    @functools.partial(
        pl.kernel,
        mesh=mesh,
        out_type=jax.ShapeDtypeStruct((b, l, hid), jnp.float32),
        scratch_types=[
            pltpu.VMEM((bp_w, LPAD), jnp.int32),
            pltpu.VMEM((NBUF, LPAD, hid), jnp.float32),
            pltpu.SemaphoreType.DMA((NBUF,)),
            pltpu.SemaphoreType.DMA((NBUF,)),
        ],
        compiler_params=pltpu.CompilerParams(use_tc_tiling_on_sc=False),
    )
    def k(idx_hbm, table_hbm, out_hbm, idx_v, rows_v, gsem, osem):
        wid = lax.axis_index("s") * info.num_cores + lax.axis_index("c")
        base_b = wid * bp_w
        pltpu.sync_copy(idx_hbm.at[pl.ds(base_b, bp_w)], idx_v)

        def start_gather(j, s):
            pltpu.make_async_copy(table_hbm.at[idx_v.at[j]], rows_v.at[s],
                                  gsem.at[s]).start()

        for s in range(NBUF):
            start_gather(s, s)

        def outer(g, carry):
            for s in range(NBUF):
                j = g * NBUF + s
                pltpu.make_async_copy(table_hbm.at[idx_v.at[0]],
                                      rows_v.at[s], gsem.at[s]).wait()
                out_copy = pltpu.make_async_copy(
                    rows_v.at[s].at[pl.ds(0, l)],
                    out_hbm.at[base_b + j], osem.at[s])
                out_copy.start()
                out_copy.wait()

                @pl.when(j + NBUF < bp_w)
                def _():
                    start_gather(j + NBUF, s)

            return carry

        lax.fori_loop(0, n_outer, outer, 0)

    return k


def kernel(seq, len_seq, item_embeddings):
    b, l = seq.shape
    seq = seq.astype(jnp.int32)
    idx_pad = jnp.concatenate(
        [seq, jnp.broadcast_to(seq[:, :1], (b, LPAD - l))], axis=1)
    return _gather_grid(b, l, HID)(idx_pad, item_embeddings)


# speedup vs baseline: 1.5423x; 1.5423x over previous
import functools

import jax
import jax.numpy as jnp
from jax import lax
from jax.experimental import pallas as pl
from jax.experimental.pallas import tpu as pltpu
from jax.experimental.pallas import tpu_sc as plsc

HID = 64
PAD = 128
LPAD = 56
IPAD = 128
NBUF = 8


def _gather_grid(b: int, l: int, hid: int, vocab: int):
    info = plsc.get_sparse_core_info()
    num_workers = info.num_cores * info.num_subcores
    bp_w = b // num_workers
    n_outer = bp_w // NBUF
    mesh = plsc.VectorSubcoreMesh(core_axis_name="c", subcore_axis_name="s")

    @functools.partial(
        pl.kernel,
        mesh=mesh,
        out_type=jax.ShapeDtypeStruct((b, LPAD, PAD), jnp.float32),
        scratch_types=[
            pltpu.VMEM((bp_w, IPAD), jnp.int32),
            pltpu.VMEM((NBUF, LPAD, hid), jnp.float32),
            pltpu.SemaphoreType.DMA((NBUF,)),
            pltpu.SemaphoreType.DMA((NBUF,)),
        ],
        compiler_params=pltpu.CompilerParams(use_tc_tiling_on_sc=False),
    )
    def k(idx_hbm, table_hbm, out_hbm, idx_v, rows_v, gsem, osem):
        wid = lax.axis_index("s") * info.num_cores + lax.axis_index("c")
        base_b = wid * bp_w
        pltpu.sync_copy(idx_hbm.at[pl.ds(base_b, bp_w)], idx_v)

        def start_gather(j, s):
            idx_c = idx_v.at[j].at[pl.ds(0, LPAD)]
            pltpu.make_async_copy(table_hbm.at[idx_c], rows_v.at[s],
                                  gsem.at[s]).start()

        for s in range(NBUF):
            start_gather(s, s)

        def outer(g, carry):
            for s in range(NBUF):
                j = g * NBUF + s
                pltpu.make_async_copy(
                    table_hbm.at[idx_v.at[0].at[pl.ds(0, LPAD)]],
                    rows_v.at[s], gsem.at[s]).wait()
                out_copy = pltpu.make_async_copy(
                    rows_v.at[s],
                    out_hbm.at[base_b + j].at[:, pl.ds(0, hid)],
                    osem.at[s])
                out_copy.start()
                out_copy.wait()

                @pl.when(j + NBUF < bp_w)
                def _():
                    start_gather(j + NBUF, s)

            return carry

        lax.fori_loop(0, n_outer, outer, 0)

    return k


def kernel(seq, len_seq, item_embeddings):
    b, l = seq.shape
    vocab = item_embeddings.shape[0]
    seq = seq.astype(jnp.int32)
    idx2 = jnp.concatenate(
        [seq, jnp.broadcast_to(seq[:, :1], (b, IPAD - l))], axis=1) * 2
    table2 = jnp.pad(item_embeddings,
                     ((0, 0), (0, PAD - HID))).reshape(2 * vocab, HID)
    out_padded = _gather_grid(b, l, HID, vocab)(idx2, table2)
    return out_padded[:, :l, :HID]
